# Initial kernel scaffold; baseline (speedup 1.0000x reference)
import jax
import jax.numpy as jnp
from jax.experimental import pallas as pl


def kernel(heatmap):
    raise NotImplementedError("write your pallas kernel here")


# speedup vs baseline: 1.2254x; 1.2254x over previous
import jax
import jax.numpy as jnp
from jax import lax
from jax.experimental import pallas as pl

WINDOW_SIZE = 16
TEMPERATURE = 0.01


def _softargmax_kernel(hm_ref, out_ref):
    H, W = hm_ref.shape[1], hm_ref.shape[2]
    hm = hm_ref[0]

    m = jnp.max(hm)
    rows = lax.broadcasted_iota(jnp.int32, (H, W), 0)
    cols = lax.broadcasted_iota(jnp.int32, (H, W), 1)
    flat = rows * W + cols
    idx = jnp.min(jnp.where(hm == m, flat, jnp.int32(H * W)))
    y0 = idx // W
    x0 = idx - y0 * W

    half = WINDOW_SIZE // 2
    xmin = jnp.maximum(x0 - half, 0)
    xmax = jnp.minimum(x0 + half, W)
    ymin = jnp.maximum(y0 - half, 0)
    ymax = jnp.minimum(y0 + half, H)

    BAND = WINDOW_SIZE + 8
    ystart = jnp.minimum((jnp.maximum(y0 - half, 0) // 8) * 8, H - BAND)
    band = hm_ref[0, pl.ds(ystart, BAND), :]

    gy = ystart + lax.broadcasted_iota(jnp.int32, (BAND, W), 0)
    gx = lax.broadcasted_iota(jnp.int32, (BAND, W), 1)
    mask = (gx >= xmin) & (gx < xmax) & (gy >= ymin) & (gy < ymax)

    e = jnp.where(mask, jnp.exp((band - m) * (1.0 / TEMPERATURE)), 0.0)
    s = jnp.sum(e)
    p = e / s

    fx = gx.astype(jnp.float32)
    fy = gy.astype(jnp.float32)
    x_mean = jnp.sum(fx * p)
    y_mean = jnp.sum(fy * p)
    dx = fx - x_mean
    dy = fy - y_mean
    var_xx = jnp.sum(p * dx * dx)
    var_yy = jnp.sum(p * dy * dy)
    cov_xy = jnp.sum(p * dx * dy)

    vals = jnp.concatenate(
        [
            (x_mean * (1.0 / (W - 1))).reshape(1, 1),
            (y_mean * (1.0 / (H - 1))).reshape(1, 1),
            var_xx.reshape(1, 1),
            cov_xy.reshape(1, 1),
            cov_xy.reshape(1, 1),
            var_yy.reshape(1, 1),
            (var_xx + var_yy).reshape(1, 1),
            jnp.zeros((1, 1), jnp.float32),
        ],
        axis=1,
    )
    out_ref[0] = vals


def kernel(heatmap):
    B, C, H, W = heatmap.shape
    n = B * C
    hm = heatmap.reshape(n, H, W)

    stats = pl.pallas_call(
        _softargmax_kernel,
        grid=(n,),
        in_specs=[pl.BlockSpec((1, H, W), lambda i: (i, 0, 0))],
        out_specs=pl.BlockSpec((1, 1, 8), lambda i: (i, 0, 0)),
        out_shape=jax.ShapeDtypeStruct((n, 1, 8), jnp.float32),
    )(hm)

    stats = stats.reshape(n, 8)
    coords = stats[:, 0:2].reshape(B, C, 2)
    cov = stats[:, 2:6].reshape(B, C, 2, 2)
    spread = stats[:, 6:7].reshape(B, C, 1)
    return (coords, cov, spread)


# speedup vs baseline: 1.6611x; 1.3556x over previous
import jax
import jax.numpy as jnp
from jax import lax
from jax.experimental import pallas as pl

WINDOW_SIZE = 16
TEMPERATURE = 0.01


def _softargmax_kernel(hm_ref, out_ref):
    K, H, W = hm_ref.shape
    half = WINDOW_SIZE // 2
    BAND = WINDOW_SIZE + 8

    rows = lax.broadcasted_iota(jnp.int32, (H, W), 0)
    cols = lax.broadcasted_iota(jnp.int32, (H, W), 1)
    flat = rows * W + cols
    bgy = lax.broadcasted_iota(jnp.int32, (BAND, W), 0)
    bgx = lax.broadcasted_iota(jnp.int32, (BAND, W), 1)

    for j in range(K):
        hm = hm_ref[j]

        m = jnp.max(hm)
        idx = jnp.min(jnp.where(hm == m, flat, jnp.int32(H * W)))
        y0 = idx // W
        x0 = idx - y0 * W

        xmin = jnp.maximum(x0 - half, 0)
        xmax = jnp.minimum(x0 + half, W)
        ymin = jnp.maximum(y0 - half, 0)
        ymax = jnp.minimum(y0 + half, H)

        ystart = jnp.minimum((jnp.maximum(y0 - half, 0) // 8) * 8, H - BAND)
        band = hm_ref[j, pl.ds(ystart, BAND), :]

        gy = ystart + bgy
        mask = (bgx >= xmin) & (bgx < xmax) & (gy >= ymin) & (gy < ymax)

        e = jnp.where(mask, jnp.exp((band - m) * (1.0 / TEMPERATURE)), 0.0)
        s = jnp.sum(e)
        p = e / s

        fx = bgx.astype(jnp.float32)
        fy = gy.astype(jnp.float32)
        x_mean = jnp.sum(fx * p)
        y_mean = jnp.sum(fy * p)
        dx = fx - x_mean
        dy = fy - y_mean
        var_xx = jnp.sum(p * dx * dx)
        var_yy = jnp.sum(p * dy * dy)
        cov_xy = jnp.sum(p * dx * dy)

        vals = jnp.concatenate(
            [
                (x_mean * (1.0 / (W - 1))).reshape(1, 1),
                (y_mean * (1.0 / (H - 1))).reshape(1, 1),
                var_xx.reshape(1, 1),
                cov_xy.reshape(1, 1),
                cov_xy.reshape(1, 1),
                var_yy.reshape(1, 1),
                (var_xx + var_yy).reshape(1, 1),
                jnp.zeros((1, 1), jnp.float32),
            ],
            axis=1,
        )
        out_ref[j] = vals


def kernel(heatmap):
    B, C, H, W = heatmap.shape
    n = B * C
    hm = heatmap.reshape(n, H, W)

    K = 8
    assert n % K == 0
    stats = pl.pallas_call(
        _softargmax_kernel,
        grid=(n // K,),
        in_specs=[pl.BlockSpec((K, H, W), lambda i: (i, 0, 0))],
        out_specs=pl.BlockSpec((K, 1, 8), lambda i: (i, 0, 0)),
        out_shape=jax.ShapeDtypeStruct((n, 1, 8), jnp.float32),
    )(hm)

    stats = stats.reshape(n, 8)
    coords = stats[:, 0:2].reshape(B, C, 2)
    cov = stats[:, 2:6].reshape(B, C, 2, 2)
    spread = stats[:, 6:7].reshape(B, C, 1)
    return (coords, cov, spread)


# speedup vs baseline: 6.1336x; 3.6925x over previous
import jax
import jax.numpy as jnp
from jax import lax
from jax.experimental import pallas as pl

WINDOW_SIZE = 16
TEMPERATURE = 0.01


def _softargmax_kernel(hm_ref, out_ref):
    K, H, W = hm_ref.shape
    half = WINDOW_SIZE // 2
    BAND = WINDOW_SIZE + 8

    hm = hm_ref[...]

    m_v = jnp.max(hm, axis=(1, 2))
    rows = lax.broadcasted_iota(jnp.int32, (K, H, W), 1)
    cols = lax.broadcasted_iota(jnp.int32, (K, H, W), 2)
    flat = rows * W + cols
    idx_v = jnp.min(
        jnp.where(hm == m_v[:, None, None], flat, jnp.int32(H * W)),
        axis=(1, 2),
    )
    y0_v = idx_v // W
    x0_v = idx_v - y0_v * W

    xmin_v = jnp.maximum(x0_v - half, 0)
    xmax_v = jnp.minimum(x0_v + half, W)
    ymin_v = jnp.maximum(y0_v - half, 0)
    ymax_v = jnp.minimum(y0_v + half, H)

    ystart_v = jnp.minimum((jnp.maximum(y0_v - half, 0) // 8) * 8, H - BAND)

    bands = jnp.stack(
        [
            hm_ref[j, pl.ds(pl.multiple_of(ystart_v[j], 8), BAND), :]
            for j in range(K)
        ]
    )

    bgy = lax.broadcasted_iota(jnp.int32, (K, BAND, W), 1)
    bgx = lax.broadcasted_iota(jnp.int32, (K, BAND, W), 2)
    gy = ystart_v[:, None, None] + bgy
    mask = (
        (bgx >= xmin_v[:, None, None])
        & (bgx < xmax_v[:, None, None])
        & (gy >= ymin_v[:, None, None])
        & (gy < ymax_v[:, None, None])
    )

    e = jnp.where(
        mask, jnp.exp((bands - m_v[:, None, None]) * (1.0 / TEMPERATURE)), 0.0
    )
    s = jnp.sum(e, axis=(1, 2))
    p = e / s[:, None, None]

    fx = bgx.astype(jnp.float32)
    fy = gy.astype(jnp.float32)
    x_mean = jnp.sum(fx * p, axis=(1, 2))
    y_mean = jnp.sum(fy * p, axis=(1, 2))
    dx = fx - x_mean[:, None, None]
    dy = fy - y_mean[:, None, None]
    var_xx = jnp.sum(p * dx * dx, axis=(1, 2))
    var_yy = jnp.sum(p * dy * dy, axis=(1, 2))
    cov_xy = jnp.sum(p * dx * dy, axis=(1, 2))

    out_ref[...] = jnp.stack(
        [
            x_mean * (1.0 / (W - 1)),
            y_mean * (1.0 / (H - 1)),
            var_xx,
            cov_xy,
            cov_xy,
            var_yy,
            var_xx + var_yy,
            jnp.zeros((K,), jnp.float32),
        ],
        axis=-1,
    )


def kernel(heatmap):
    B, C, H, W = heatmap.shape
    n = B * C
    hm = heatmap.reshape(n, H, W)

    K = 8
    assert n % K == 0
    stats = pl.pallas_call(
        _softargmax_kernel,
        grid=(n // K,),
        in_specs=[pl.BlockSpec((K, H, W), lambda i: (i, 0, 0))],
        out_specs=pl.BlockSpec((K, 8), lambda i: (i, 0)),
        out_shape=jax.ShapeDtypeStruct((n, 8), jnp.float32),
    )(hm)

    coords = stats[:, 0:2].reshape(B, C, 2)
    cov = stats[:, 2:6].reshape(B, C, 2, 2)
    spread = stats[:, 6:7].reshape(B, C, 1)
    return (coords, cov, spread)


# speedup vs baseline: 8.0765x; 1.3168x over previous
import jax
import jax.numpy as jnp
from jax import lax
from jax.experimental import pallas as pl

WINDOW_SIZE = 16
TEMPERATURE = 0.01


def _softargmax_kernel(hm_ref, out_ref):
    K, H, W = hm_ref.shape
    half = WINDOW_SIZE // 2
    BAND = WINDOW_SIZE + 8

    hm = hm_ref[...]

    m_v = jnp.max(hm, axis=(1, 2))
    rows = lax.broadcasted_iota(jnp.int32, (K, H, W), 1)
    cols = lax.broadcasted_iota(jnp.int32, (K, H, W), 2)
    flat = rows * W + cols
    idx_v = jnp.min(
        jnp.where(hm == m_v[:, None, None], flat, jnp.int32(H * W)),
        axis=(1, 2),
    )
    y0_v = idx_v // W
    x0_v = idx_v - y0_v * W

    xmin_v = jnp.maximum(x0_v - half, 0)
    xmax_v = jnp.minimum(x0_v + half, W)
    ymin_v = jnp.maximum(y0_v - half, 0)
    ymax_v = jnp.minimum(y0_v + half, H)

    ystart_v = jnp.minimum((jnp.maximum(y0_v - half, 0) // 8) * 8, H - BAND)

    bands = jnp.stack(
        [
            hm_ref[j, pl.ds(pl.multiple_of(ystart_v[j], 8), BAND), :]
            for j in range(K)
        ]
    )

    bgy = lax.broadcasted_iota(jnp.int32, (K, BAND, W), 1)
    bgx = lax.broadcasted_iota(jnp.int32, (K, BAND, W), 2)
    gy = ystart_v[:, None, None] + bgy
    mask = (
        (bgx >= xmin_v[:, None, None])
        & (bgx < xmax_v[:, None, None])
        & (gy >= ymin_v[:, None, None])
        & (gy < ymax_v[:, None, None])
    )

    e = jnp.where(
        mask, jnp.exp((bands - m_v[:, None, None]) * (1.0 / TEMPERATURE)), 0.0
    )
    s = jnp.sum(e, axis=(1, 2))
    p = e / s[:, None, None]

    fx = bgx.astype(jnp.float32)
    fy = gy.astype(jnp.float32)
    x_mean = jnp.sum(fx * p, axis=(1, 2))
    y_mean = jnp.sum(fy * p, axis=(1, 2))
    dx = fx - x_mean[:, None, None]
    dy = fy - y_mean[:, None, None]
    var_xx = jnp.sum(p * dx * dx, axis=(1, 2))
    var_yy = jnp.sum(p * dy * dy, axis=(1, 2))
    cov_xy = jnp.sum(p * dx * dy, axis=(1, 2))

    out_ref[...] = jnp.stack(
        [
            x_mean * (1.0 / (W - 1)),
            y_mean * (1.0 / (H - 1)),
            var_xx,
            cov_xy,
            cov_xy,
            var_yy,
            var_xx + var_yy,
            jnp.zeros((K,), jnp.float32),
        ],
        axis=-1,
    )


def kernel(heatmap):
    B, C, H, W = heatmap.shape
    n = B * C
    hm = heatmap.reshape(n, H, W)

    K = 16
    assert n % K == 0
    stats = pl.pallas_call(
        _softargmax_kernel,
        grid=(n // K,),
        in_specs=[pl.BlockSpec((K, H, W), lambda i: (i, 0, 0))],
        out_specs=pl.BlockSpec((K, 8), lambda i: (i, 0)),
        out_shape=jax.ShapeDtypeStruct((n, 8), jnp.float32),
    )(hm)

    coords = stats[:, 0:2].reshape(B, C, 2)
    cov = stats[:, 2:6].reshape(B, C, 2, 2)
    spread = stats[:, 6:7].reshape(B, C, 1)
    return (coords, cov, spread)


# speedup vs baseline: 9.1457x; 1.1324x over previous
import jax
import jax.numpy as jnp
from jax import lax
from jax.experimental import pallas as pl

WINDOW_SIZE = 16
TEMPERATURE = 0.01


def _softargmax_kernel(hm_ref, out_ref):
    K, H, W = hm_ref.shape
    half = WINDOW_SIZE // 2
    BAND = WINDOW_SIZE + 8

    hm = hm_ref[...]

    m_v = jnp.max(hm, axis=(1, 2))
    rows = lax.broadcasted_iota(jnp.int32, (K, H, W), 1)
    cols = lax.broadcasted_iota(jnp.int32, (K, H, W), 2)
    flat = rows * W + cols
    idx_v = jnp.min(
        jnp.where(hm == m_v[:, None, None], flat, jnp.int32(H * W)),
        axis=(1, 2),
    )
    y0_v = idx_v // W
    x0_v = idx_v - y0_v * W

    xmin_v = jnp.maximum(x0_v - half, 0)
    xmax_v = jnp.minimum(x0_v + half, W)
    ymin_v = jnp.maximum(y0_v - half, 0)
    ymax_v = jnp.minimum(y0_v + half, H)

    ystart_v = jnp.minimum((jnp.maximum(y0_v - half, 0) // 8) * 8, H - BAND)

    bands = jnp.stack(
        [
            hm_ref[j, pl.ds(pl.multiple_of(ystart_v[j], 8), BAND), :]
            for j in range(K)
        ]
    )

    bgy = lax.broadcasted_iota(jnp.int32, (K, BAND, W), 1)
    bgx = lax.broadcasted_iota(jnp.int32, (K, BAND, W), 2)
    gy = ystart_v[:, None, None] + bgy
    mask = (
        (bgx >= xmin_v[:, None, None])
        & (bgx < xmax_v[:, None, None])
        & (gy >= ymin_v[:, None, None])
        & (gy < ymax_v[:, None, None])
    )

    e = jnp.where(
        mask, jnp.exp((bands - m_v[:, None, None]) * (1.0 / TEMPERATURE)), 0.0
    )
    s = jnp.sum(e, axis=(1, 2))
    p = e / s[:, None, None]

    fx = bgx.astype(jnp.float32)
    fy = gy.astype(jnp.float32)
    x_mean = jnp.sum(fx * p, axis=(1, 2))
    y_mean = jnp.sum(fy * p, axis=(1, 2))
    dx = fx - x_mean[:, None, None]
    dy = fy - y_mean[:, None, None]
    var_xx = jnp.sum(p * dx * dx, axis=(1, 2))
    var_yy = jnp.sum(p * dy * dy, axis=(1, 2))
    cov_xy = jnp.sum(p * dx * dy, axis=(1, 2))

    out_ref[...] = jnp.stack(
        [
            x_mean * (1.0 / (W - 1)),
            y_mean * (1.0 / (H - 1)),
            var_xx,
            cov_xy,
            cov_xy,
            var_yy,
            var_xx + var_yy,
            jnp.zeros((K,), jnp.float32),
        ],
        axis=-1,
    )


def kernel(heatmap):
    B, C, H, W = heatmap.shape
    n = B * C
    hm = heatmap.reshape(n, H, W)

    K = 32
    assert n % K == 0
    stats = pl.pallas_call(
        _softargmax_kernel,
        grid=(n // K,),
        in_specs=[pl.BlockSpec((K, H, W), lambda i: (i, 0, 0))],
        out_specs=pl.BlockSpec((K, 8), lambda i: (i, 0)),
        out_shape=jax.ShapeDtypeStruct((n, 8), jnp.float32),
    )(hm)

    coords = stats[:, 0:2].reshape(B, C, 2)
    cov = stats[:, 2:6].reshape(B, C, 2, 2)
    spread = stats[:, 6:7].reshape(B, C, 1)
    return (coords, cov, spread)
